# baseline (speedup 1.0000x reference)
import functools

import jax
import jax.numpy as jnp
from jax import lax
from jax.experimental import pallas as pl
from jax.experimental.pallas import tpu as pltpu
from jax.experimental.pallas import tpu_sc as plsc

VOCAB = 100000
MAX_POS = 2048
HIDDEN = 768
VIS_DIM = 2048
B = 4
S = 2048
NREG = 36
LN_EPS = 1e-5

_info = plsc.get_sparse_core_info()
NC, NS, L = _info.num_cores, _info.num_subcores, _info.num_lanes
NW = NC * NS
ROWS = B * S
ROWS_PER_W = ROWS // NW
CHUNK = 32
CHUNKS = ROWS_PER_W // CHUNK
NBW = 3
NBP = 2
PF = 1
H16 = HIDDEN // 16


def _emb_body(ids_hbm, wtab_hbm, ptab_hbm, out_hbm,
              idx_t, idx_p,
              bw0, bw1, bw2, bp0, bp1,
              sg0, sg1, sg2, sp0, sp1, ss0, ss1, ss2, sgi):
    bw = (bw0, bw1, bw2)
    bp = (bp0, bp1)
    sg = (sg0, sg1, sg2)
    sp = (sp0, sp1)
    ss = (ss0, ss1, ss2)
    wid = lax.axis_index("s") * NC + lax.axis_index("c")
    base = wid * ROWS_PER_W
    ci_t = pltpu.async_copy(ids_hbm.at[0, wid], idx_t, sgi)
    ci_p = pltpu.async_copy(ids_hbm.at[1, wid], idx_p, sgi)
    ci_t.wait()
    ci_p.wait()

    gath = [None] * CHUNKS
    scat = [None] * CHUNKS
    drained = set()

    def start_gather(c):
        gath[c] = (pltpu.async_copy(wtab_hbm.at[idx_t.at[c]], bw[c % NBW], sg[c % NBW]),
                   pltpu.async_copy(ptab_hbm.at[idx_p.at[c]], bp[c % NBP], sp[c % NBP]))

    for c in range(PF):
        start_gather(c)
    for c in range(CHUNKS):
        sw = c % NBW
        if c + PF < CHUNKS:
            prev = c + PF - NBW
            if prev >= 0:
                for sc_ in scat[prev]:
                    sc_.wait()
                drained.add(prev)
            start_gather(c + PF)
        gw, gp = gath[c]
        gw.wait()
        gp.wait()

        def add_row(r, carry, sw=sw, spi=c % NBP):
            for j in range(H16):
                col = j * L
                plsc.addupdate(bw[sw].at[r, pl.ds(col, L)], bp[spi][r, pl.ds(col, L)])
            return carry

        H = CHUNK // 2
        lax.fori_loop(0, H, add_row, 0)
        s1 = pltpu.async_copy(bw[sw].at[pl.ds(0, H)],
                              out_hbm.at[pl.ds(base + c * CHUNK, H)], ss[sw])
        lax.fori_loop(H, CHUNK, add_row, 0)
        s2 = pltpu.async_copy(bw[sw].at[pl.ds(H, H)],
                              out_hbm.at[pl.ds(base + c * CHUNK + H, H)], ss[sw])
        scat[c] = (s1, s2)
    for c in range(CHUNKS):
        if c not in drained:
            for sc_ in scat[c]:
                sc_.wait()


_emb = functools.partial(
    pl.kernel,
    mesh=plsc.VectorSubcoreMesh(core_axis_name="c", subcore_axis_name="s"),
    out_type=jax.ShapeDtypeStruct((ROWS, HIDDEN), jnp.float32),
    scratch_types=(
        [pltpu.VMEM((CHUNKS, CHUNK), jnp.int32)] * 2
        + [pltpu.VMEM((CHUNK, HIDDEN), jnp.float32)] * (NBW + NBP)
        + [pltpu.SemaphoreType.DMA] * (NBW + NBP + NBW + 1)
    ),
)(_emb_body)


def _visn_body(x_ref, w_ref, b_ref, g_ref, bt_ref, o_ref):
    x = x_ref[...]
    w = w_ref[...]
    v = jnp.dot(x, w, preferred_element_type=jnp.float32) + b_ref[...]
    mean = jnp.mean(v, axis=1, keepdims=True)
    d = v - mean
    var = jnp.mean(d * d, axis=1, keepdims=True)
    o_ref[...] = d * lax.rsqrt(var + LN_EPS) * g_ref[...] + bt_ref[...]


_visn = pl.pallas_call(
    _visn_body,
    out_shape=jax.ShapeDtypeStruct((B * NREG, HIDDEN), jnp.float32),
)


def kernel(token_ids, image_feat, position_ids, word_emb, pos_emb,
           visn_W, visn_b, ln_gamma, ln_beta):
    ids = jnp.stack([token_ids.astype(jnp.int32), position_ids.astype(jnp.int32)])
    ids = ids.reshape(2, NW, CHUNKS, CHUNK)
    emb = _emb(ids, word_emb, pos_emb).reshape(B, S, HIDDEN)
    v = _visn(image_feat.reshape(B * NREG, VIS_DIM), visn_W,
              visn_b.reshape(1, HIDDEN), ln_gamma.reshape(1, HIDDEN),
              ln_beta.reshape(1, HIDDEN)).reshape(B, NREG, HIDDEN)
    return (emb, v)


# speedup vs baseline: 1.0703x; 1.0703x over previous
import functools

import jax
import jax.numpy as jnp
from jax import lax
from jax.experimental import pallas as pl
from jax.experimental.pallas import tpu as pltpu
from jax.experimental.pallas import tpu_sc as plsc

VOCAB = 100000
MAX_POS = 2048
HIDDEN = 768
VIS_DIM = 2048
B = 4
S = 2048
NREG = 36
LN_EPS = 1e-5

_info = plsc.get_sparse_core_info()
NC, NS, L = _info.num_cores, _info.num_subcores, _info.num_lanes
NW = NC * NS
ROWS = B * S
ROWS_PER_W = ROWS // NW
CHUNK = 32
CHUNKS = ROWS_PER_W // CHUNK
NBW = 3
NBP = 2
PF = 1
H16 = HIDDEN // 16


def _emb_body(ids_hbm, wtab_hbm, ptab_hbm, out_hbm,
              idx_t, idx_p,
              bw0, bw1, bw2, bp0, bp1,
              sg0, sg1, sg2, sp0, sp1, ss0, ss1, ss2, sgi):
    bw = (bw0, bw1, bw2)
    bp = (bp0, bp1)
    sg = (sg0, sg1, sg2)
    sp = (sp0, sp1)
    ss = (ss0, ss1, ss2)
    wid = lax.axis_index("s") * NC + lax.axis_index("c")
    base = wid * ROWS_PER_W
    ci_t = pltpu.async_copy(ids_hbm.at[0, wid], idx_t, sgi)
    ci_p = pltpu.async_copy(ids_hbm.at[1, wid], idx_p, sgi)
    ci_t.wait()
    ci_p.wait()

    gath = [None] * CHUNKS
    scat = [None] * CHUNKS
    drained = set()

    def start_gather(c):
        gath[c] = (pltpu.async_copy(wtab_hbm.at[idx_t.at[c]], bw[c % NBW], sg[c % NBW]),
                   pltpu.async_copy(ptab_hbm.at[idx_p.at[c]], bp[c % NBP], sp[c % NBP]))

    for c in range(PF):
        start_gather(c)
    for c in range(CHUNKS):
        sw = c % NBW
        if c + PF < CHUNKS:
            prev = c + PF - NBW
            if prev >= 0:
                for sc_ in scat[prev]:
                    sc_.wait()
                drained.add(prev)
            start_gather(c + PF)
        gw, gp = gath[c]
        gw.wait()
        gp.wait()

        def add_row(r, carry, sw=sw, spi=c % NBP):
            for j in range(H16):
                col = j * L
                plsc.addupdate(bw[sw].at[r, pl.ds(col, L)], bp[spi][r, pl.ds(col, L)])
            return carry

        lax.fori_loop(0, CHUNK, add_row, 0, unroll=2)
        scat[c] = (pltpu.async_copy(bw[sw], out_hbm.at[pl.ds(base + c * CHUNK, CHUNK)], ss[sw]),)
    for c in range(CHUNKS):
        if c not in drained:
            for sc_ in scat[c]:
                sc_.wait()


_emb = functools.partial(
    pl.kernel,
    mesh=plsc.VectorSubcoreMesh(core_axis_name="c", subcore_axis_name="s"),
    out_type=jax.ShapeDtypeStruct((ROWS, HIDDEN), jnp.float32),
    scratch_types=(
        [pltpu.VMEM((CHUNKS, CHUNK), jnp.int32)] * 2
        + [pltpu.VMEM((CHUNK, HIDDEN), jnp.float32)] * (NBW + NBP)
        + [pltpu.SemaphoreType.DMA] * (NBW + NBP + NBW + 1)
    ),
)(_emb_body)


def _visn_body(x_ref, w_ref, b_ref, g_ref, bt_ref, o_ref):
    x = x_ref[...]
    w = w_ref[...]
    v = jnp.dot(x, w, preferred_element_type=jnp.float32) + b_ref[...]
    mean = jnp.mean(v, axis=1, keepdims=True)
    d = v - mean
    var = jnp.mean(d * d, axis=1, keepdims=True)
    o_ref[...] = d * lax.rsqrt(var + LN_EPS) * g_ref[...] + bt_ref[...]


_visn = pl.pallas_call(
    _visn_body,
    out_shape=jax.ShapeDtypeStruct((B * NREG, HIDDEN), jnp.float32),
)


def kernel(token_ids, image_feat, position_ids, word_emb, pos_emb,
           visn_W, visn_b, ln_gamma, ln_beta):
    ids = jnp.stack([token_ids.astype(jnp.int32), position_ids.astype(jnp.int32)])
    ids = ids.reshape(2, NW, CHUNKS, CHUNK)
    emb = _emb(ids, word_emb, pos_emb).reshape(B, S, HIDDEN)
    v = _visn(image_feat.reshape(B * NREG, VIS_DIM), visn_W,
              visn_b.reshape(1, HIDDEN), ln_gamma.reshape(1, HIDDEN),
              ln_beta.reshape(1, HIDDEN)).reshape(B, NREG, HIDDEN)
    return (emb, v)
